# baseline (speedup 1.0000x reference)
import functools

import jax
import jax.numpy as jnp
from jax import lax
from jax.experimental import pallas as pl
from jax.experimental.pallas import tpu as pltpu
from jax.experimental.pallas import tpu_sc as plsc

CHUNK = 128
LANES = 16


def _tta_kernel(n_rows, dim, vocab, num_cores, num_subcores):
    n_workers = num_cores * num_subcores
    per_w = n_rows // n_workers
    n_chunks = per_w // CHUNK
    mesh = plsc.VectorSubcoreMesh(core_axis_name="c", subcore_axis_name="s")

    @functools.partial(
        pl.kernel,
        mesh=mesh,
        out_type=jax.ShapeDtypeStruct((n_rows, dim), jnp.float32),
        compiler_params=pltpu.CompilerParams(use_tc_tiling_on_sc=False),
        scratch_types=[
            pltpu.VMEM((CHUNK,), jnp.int32),
            pltpu.VMEM((CHUNK, dim), jnp.float32),
            pltpu.VMEM((CHUNK, dim), jnp.float32),
            pltpu.SemaphoreType.DMA,
        ],
    )
    def k(x_hbm, idx_hbm, tab_hbm, out_hbm, idx_v, acc_v, xb_v, sem):
        wid = lax.axis_index("s") * num_cores + lax.axis_index("c")
        base = wid * per_w

        def chunk_body(c, carry):
            row0 = base + c * CHUNK
            pltpu.sync_copy(idx_hbm.at[pl.ds(row0, CHUNK)], idx_v)
            for j in range(CHUNK // LANES):
                t = idx_v[pl.ds(j * LANES, LANES)] - 1
                t = jnp.where(t < 0, t + vocab, t)
                idx_v[pl.ds(j * LANES, LANES)] = t
            pltpu.sync_copy(x_hbm.at[pl.ds(row0, CHUNK)], xb_v)
            pltpu.async_copy(tab_hbm.at[idx_v], xb_v, sem, add=True).wait()
            pltpu.sync_copy(xb_v, out_hbm.at[pl.ds(row0, CHUNK)])
            return carry

        lax.fori_loop(0, n_chunks, chunk_body, 0, unroll=False)

    return k


def kernel(x, tta, embedding):
    b, h, d = x.shape
    vocab = embedding.shape[0]
    n_rows = b * h
    x2 = x.reshape(n_rows, d)
    idx = tta.reshape(n_rows).astype(jnp.int32)
    info = plsc.get_sparse_core_info()
    k = _tta_kernel(n_rows, d, vocab, info.num_cores, info.num_subcores)
    out = k(x2, idx, embedding)
    return out.reshape(b, h, d)


# speedup vs baseline: 1.3080x; 1.3080x over previous
import functools

import jax
import jax.numpy as jnp
from jax import lax
from jax.experimental import pallas as pl
from jax.experimental.pallas import tpu as pltpu
from jax.experimental.pallas import tpu_sc as plsc

CHUNK = 256
NBUF = 4
LANES = 16
IDX_W = 128


def _tta_kernel(n_rows, dim, vocab, num_cores, num_subcores):
    n_workers = num_cores * num_subcores
    per_w = n_rows // n_workers
    n_chunks = per_w // CHUNK
    n_groups = n_chunks // NBUF
    ipc = CHUNK // IDX_W
    mesh = plsc.VectorSubcoreMesh(core_axis_name="c", subcore_axis_name="s")

    @functools.partial(
        pl.kernel,
        mesh=mesh,
        out_type=jax.ShapeDtypeStruct((n_rows, dim), jnp.float32),
        compiler_params=pltpu.CompilerParams(use_tc_tiling_on_sc=False),
        scratch_types=(
            [pltpu.VMEM((ipc, IDX_W), jnp.int32)] * NBUF
            + [pltpu.VMEM((CHUNK, dim), jnp.float32)] * NBUF
            + [pltpu.SemaphoreType.DMA] * (3 * NBUF)
        ),
    )
    def k(x_hbm, idx_hbm, tab_hbm, out_hbm, *scr):
        idx_v = scr[0:NBUF]
        xb_v = scr[NBUF : 2 * NBUF]
        in_sem = scr[2 * NBUF : 3 * NBUF]
        g_sem = scr[3 * NBUF : 4 * NBUF]
        out_sem = scr[4 * NBUF : 5 * NBUF]

        wid = lax.axis_index("s") * num_cores + lax.axis_index("c")
        base = wid * per_w

        def fire_in(b, c):
            row0 = base + c * CHUNK
            pltpu.async_copy(
                idx_hbm.at[pl.ds(row0 // IDX_W, ipc)], idx_v[b], in_sem[b]
            )
            pltpu.async_copy(x_hbm.at[pl.ds(row0, CHUNK)], xb_v[b], in_sem[b])

        def wait_in(b):
            pltpu.make_async_copy(
                idx_hbm.at[pl.ds(0, ipc)], idx_v[b], in_sem[b]
            ).wait()
            pltpu.make_async_copy(
                x_hbm.at[pl.ds(0, CHUNK)], xb_v[b], in_sem[b]
            ).wait()

        def wait_out(b):
            pltpu.make_async_copy(
                xb_v[b], out_hbm.at[pl.ds(0, CHUNK)], out_sem[b]
            ).wait()

        for b in range(NBUF):
            fire_in(b, b)

        def group_body(g, carry):
            c0 = g * NBUF
            for b in range(NBUF):
                wait_in(b)
                for j in range(ipc):
                    for v in range(IDX_W // LANES):
                        t = idx_v[b][j, pl.ds(v * LANES, LANES)] - 1
                        t = jnp.where(t < 0, t + vocab, t)
                        idx_v[b][j, pl.ds(v * LANES, LANES)] = t
                for j in range(ipc):
                    pltpu.async_copy(
                        tab_hbm.at[idx_v[b].at[j]],
                        xb_v[b].at[pl.ds(j * IDX_W, IDX_W)],
                        g_sem[b],
                        add=True,
                    )
            for b in range(NBUF):
                for j in range(ipc):
                    pltpu.make_async_copy(
                        tab_hbm.at[idx_v[b].at[j]],
                        xb_v[b].at[pl.ds(j * IDX_W, IDX_W)],
                        g_sem[b],
                    ).wait()
                row0 = base + (c0 + b) * CHUNK
                pltpu.async_copy(
                    xb_v[b], out_hbm.at[pl.ds(row0, CHUNK)], out_sem[b]
                )
            for b in range(NBUF):
                wait_out(b)

                @pl.when(g < n_groups - 1)
                def _():
                    fire_in(b, c0 + NBUF + b)

            return carry

        lax.fori_loop(0, n_groups, group_body, 0, unroll=False)

    return k


def kernel(x, tta, embedding):
    b, h, d = x.shape
    vocab = embedding.shape[0]
    n_rows = b * h
    x2 = x.reshape(n_rows, d)
    idx = tta.reshape(n_rows // IDX_W, IDX_W).astype(jnp.int32)
    info = plsc.get_sparse_core_info()
    k = _tta_kernel(n_rows, d, vocab, info.num_cores, info.num_subcores)
    out = k(x2, idx, embedding)
    return out.reshape(b, h, d)


# speedup vs baseline: 1.8026x; 1.3782x over previous
import functools

import jax
import jax.numpy as jnp
from jax import lax
from jax.experimental import pallas as pl
from jax.experimental.pallas import tpu as pltpu
from jax.experimental.pallas import tpu_sc as plsc

CHUNK = 64
NBUF = 4
LANES = 16
PADW = 128


def _tta_kernel(n_rows, dim, vocab, num_cores, num_subcores):
    n_workers = num_cores * num_subcores
    per_w = n_rows // n_workers
    n_chunks = per_w // CHUNK
    n_groups = n_chunks // NBUF
    mesh = plsc.VectorSubcoreMesh(core_axis_name="c", subcore_axis_name="s")

    @functools.partial(
        pl.kernel,
        mesh=mesh,
        out_type=jax.ShapeDtypeStruct((n_rows, dim), jnp.float32),
        scratch_types=(
            [pltpu.VMEM((1, CHUNK), jnp.int32)] * NBUF
            + [pltpu.VMEM((CHUNK, 64), jnp.float32)] * NBUF
            + [pltpu.VMEM((CHUNK, PADW), jnp.float32)] * NBUF
            + [pltpu.SemaphoreType.DMA] * (3 * NBUF)
        ),
    )
    def k(x_hbm, idx_hbm, tab_hbm, out_hbm, *scr):
        idx_v = scr[0:NBUF]
        xb_v = scr[NBUF : 2 * NBUF]
        gb_v = scr[2 * NBUF : 3 * NBUF]
        in_sem = scr[3 * NBUF : 4 * NBUF]
        g_sem = scr[4 * NBUF : 5 * NBUF]
        out_sem = scr[5 * NBUF : 6 * NBUF]

        wid = lax.axis_index("s") * num_cores + lax.axis_index("c")
        base = wid * per_w

        def fire_in(b, c):
            row0 = base + c * CHUNK
            pltpu.async_copy(
                idx_hbm.at[pl.ds(row0 // CHUNK, 1)], idx_v[b], in_sem[b]
            )
            pltpu.async_copy(x_hbm.at[pl.ds(row0, CHUNK)], xb_v[b], in_sem[b])

        def wait_in(b):
            pltpu.make_async_copy(
                idx_hbm.at[pl.ds(0, 1)], idx_v[b], in_sem[b]
            ).wait()
            pltpu.make_async_copy(
                x_hbm.at[pl.ds(0, CHUNK)], xb_v[b], in_sem[b]
            ).wait()

        def wait_out(b):
            pltpu.make_async_copy(
                xb_v[b], out_hbm.at[pl.ds(0, CHUNK)], out_sem[b]
            ).wait()

        for b in range(NBUF):
            fire_in(b, b)

        def group_body(g, carry):
            c0 = g * NBUF
            for b in range(NBUF):
                wait_in(b)
                for v in range(CHUNK // LANES):
                    t = idx_v[b][0, pl.ds(v * LANES, LANES)] - 1
                    t = jnp.where(t < 0, t + vocab, t)
                    idx_v[b][0, pl.ds(v * LANES, LANES)] = t
                pltpu.async_copy(
                    tab_hbm.at[idx_v[b].at[0]], gb_v[b], g_sem[b]
                )
            for b in range(NBUF):
                pltpu.make_async_copy(
                    tab_hbm.at[idx_v[b].at[0]], gb_v[b], g_sem[b]
                ).wait()

                def add_rows(i, carry2):
                    r = i * 4
                    for rr in range(4):
                        for j in range(64 // LANES):
                            plsc.addupdate(
                                xb_v[b].at[r + rr, pl.ds(j * LANES, LANES)],
                                gb_v[b][r + rr, pl.ds(j * LANES, LANES)],
                            )
                    return carry2

                lax.fori_loop(0, CHUNK // 4, add_rows, 0, unroll=False)
                row0 = base + (c0 + b) * CHUNK
                pltpu.async_copy(
                    xb_v[b], out_hbm.at[pl.ds(row0, CHUNK)], out_sem[b]
                )
            for b in range(NBUF):
                wait_out(b)

                @pl.when(g < n_groups - 1)
                def _():
                    fire_in(b, c0 + NBUF + b)

            return carry

        lax.fori_loop(0, n_groups, group_body, 0, unroll=False)

    return k


def kernel(x, tta, embedding):
    b, h, d = x.shape
    vocab = embedding.shape[0]
    n_rows = b * h
    x2 = x.reshape(n_rows, d)
    idx = tta.reshape(n_rows // CHUNK, CHUNK).astype(jnp.int32)
    tabp = jnp.pad(embedding, ((0, 0), (0, PADW - d)))
    info = plsc.get_sparse_core_info()
    k = _tta_kernel(n_rows, d, vocab, info.num_cores, info.num_subcores)
    out = k(x2, idx, tabp)
    return out.reshape(b, h, d)
